# baseline (speedup 1.0000x reference)
import functools
import math

import jax
import jax.numpy as jnp
from jax import lax
from jax.experimental import pallas as pl
from jax.experimental.pallas import tpu as pltpu
from jax.experimental.pallas import tpu_sc as plsc

_PCT = 0.1
_NC, _NS, _L = 2, 16, 16
_NW = _NC * _NS
_INT_MIN = -(2 ** 31)


def _sc_body(n_feat, k, xbits, thr_out, rowbuf, bufa, hist, totals, thrk):
    int_min = jnp.int32(_INT_MIN)
    lane = jnp.arange(_L, dtype=jnp.int32)
    ones = jnp.ones((_L,), jnp.int32)
    zeros16 = jnp.zeros((_L,), jnp.int32)
    nchunk = n_feat // _L
    rows_per_w = xbits.shape[0] // _NW

    cid = lax.axis_index("c")
    sid = lax.axis_index("s")
    wid = sid * _NC + cid

    def zero_hist():
        def zh(i, c):
            hist[pl.ds(i * _L, _L)] = zeros16
            return c
        lax.fori_loop(0, (256 * _L) // _L, zh, 0)

    def lane_reduce():
        def lr(c, carry):
            def inner(l, acc):
                return acc + hist[pl.ds(l * 256 + c * _L, _L)]
            tot = lax.fori_loop(0, _L, inner, zeros16)
            totals[pl.ds(c * _L, _L)] = tot
            return carry
        lax.fori_loop(0, 256 // _L, lr, 0)

    def bucket_scan(kprime):
        def bs(i, carry):
            acc_above, found, bsel, cnt_gt = carry
            c = 15 - i
            t = totals[pl.ds(c * _L, _L)]
            suf = lax.rev(plsc.cumsum(lax.rev(t, (0,))), (0,))
            cge = acc_above + suf
            m = (cge >= kprime).astype(jnp.int32)
            npop = jnp.sum(m)
            found_here = jnp.logical_and(found == 0, npop > 0).astype(jnp.int32)
            sufn = jnp.sum(jnp.where(lane == npop, suf, 0))
            b_here = c * _L + npop - 1
            bsel = jnp.where(found_here == 1, b_here, bsel)
            cnt_gt = jnp.where(found_here == 1, acc_above + sufn, cnt_gt)
            found = jnp.where(npop > 0, jnp.int32(1), found)
            acc_above = acc_above + jnp.sum(t)
            return acc_above, found, bsel, cnt_gt
        _, _, bsel, cnt_gt = lax.fori_loop(
            0, 256 // _L, bs,
            (jnp.int32(0), jnp.int32(0), jnp.int32(0), jnp.int32(0)))
        return bsel, cnt_gt

    def do_pass(src, dst, shift, n_cand, kprime, prefix, first, last):
        zero_hist()
        if first:
            def hs(i, c):
                v = src[pl.ds(i * _L, _L)]
                mag = v & jnp.int32(0x7FFFFFFF)
                u = jnp.where(mag == 0, int_min,
                              jnp.where(v < 0, ~v, v | int_min))
                src[pl.ds(i * _L, _L)] = u
                bucket = lax.shift_right_logical(u, shift) & jnp.int32(0xFF)
                plsc.addupdate_scatter(hist, [lane * 256 + bucket], ones)
                return c
            lax.fori_loop(0, nchunk, hs, 0)
            niter = nchunk
        else:
            def hs(i, c):
                u = src[pl.ds(i * _L, _L)]
                valid = (i * _L + lane) < n_cand
                bucket = lax.shift_right_logical(u, shift) & jnp.int32(0xFF)
                plsc.addupdate_scatter(hist, [lane * 256 + bucket], ones,
                                       mask=valid)
                return c
            niter = (n_cand + (_L - 1)) // _L
            lax.fori_loop(0, niter, hs, 0)

        lane_reduce()
        bsel, cnt_gt = bucket_scan(kprime)
        t_b = jnp.sum(jnp.where(lane == (bsel & (_L - 1)),
                                totals[pl.ds((bsel >> 4) * _L, _L)], 0))
        prefix_new = prefix | lax.shift_left(bsel, shift)
        kprime_new = kprime - cnt_gt

        if not last:
            def cp(i, off):
                v = src[pl.ds(i * _L, _L)]
                valid = (i * _L + lane) < n_cand
                b = lax.shift_right_logical(v, shift) & jnp.int32(0xFF)
                take = jnp.logical_and(valid, b == bsel)
                ti = take.astype(jnp.int32)
                pos = off + plsc.cumsum(ti) - ti
                plsc.store_scatter(dst, [pos], v, mask=take)
                return off + jnp.sum(ti)
            lax.fori_loop(0, niter, cp, jnp.int32(0))
        return prefix_new, kprime_new, t_b

    def row_fn(r, carry):
        row = wid * rows_per_w + r
        pltpu.sync_copy(xbits.at[row], rowbuf)
        n_cand = jnp.int32(n_feat)
        kprime = jnp.int32(k)
        prefix = jnp.int32(0)
        prefix, kprime, n_cand = do_pass(rowbuf, bufa, 24, n_cand, kprime,
                                         prefix, True, False)
        prefix, kprime, n_cand = do_pass(bufa, rowbuf, 16, n_cand, kprime,
                                         prefix, False, False)
        prefix, kprime, n_cand = do_pass(rowbuf, bufa, 8, n_cand, kprime,
                                         prefix, False, False)
        prefix, kprime, n_cand = do_pass(bufa, rowbuf, 0, n_cand, kprime,
                                         prefix, False, True)
        key = prefix ^ int_min
        fbits = jnp.where(key < 0, ~prefix, key)
        thrk[...] = jnp.where(lane == r, fbits, thrk[...])
        return carry

    lax.fori_loop(0, rows_per_w, row_fn, 0)
    pltpu.sync_copy(thrk, thr_out.at[wid])


def _mask_body(x_ref, t_ref, o_ref):
    x = x_ref[...]
    o_ref[...] = jnp.where(x >= t_ref[...], x, 0.0)


def kernel(x):
    n_rows, n_feat = x.shape
    k = max(1, math.ceil(n_feat * _PCT))
    xbits = lax.bitcast_convert_type(x, jnp.int32)

    mesh = plsc.VectorSubcoreMesh(core_axis_name="c", subcore_axis_name="s",
                                  num_cores=_NC, num_subcores=_NS)
    body = functools.partial(_sc_body, n_feat, k)
    thr2d = pl.kernel(
        body,
        out_type=jax.ShapeDtypeStruct((_NW, _L), jnp.int32),
        mesh=mesh,
        scratch_types=[
            pltpu.VMEM((n_feat,), jnp.int32),
            pltpu.VMEM((n_feat,), jnp.int32),
            pltpu.VMEM((256 * _L,), jnp.int32),
            pltpu.VMEM((256,), jnp.int32),
            pltpu.VMEM((_L,), jnp.int32),
        ],
        compiler_params=pltpu.CompilerParams(needs_layout_passes=False),
    )(xbits)

    rows_per_w = n_rows // _NW
    thr = lax.bitcast_convert_type(
        thr2d[:, :rows_per_w].reshape(n_rows, 1), jnp.float32)

    rb = 8
    return pl.pallas_call(
        _mask_body,
        grid=(n_rows // rb,),
        in_specs=[pl.BlockSpec((rb, n_feat), lambda i: (i, 0)),
                  pl.BlockSpec((rb, 1), lambda i: (i, 0))],
        out_specs=pl.BlockSpec((rb, n_feat), lambda i: (i, 0)),
        out_shape=jax.ShapeDtypeStruct(x.shape, x.dtype),
    )(x, thr)


# speedup vs baseline: 1.0991x; 1.0991x over previous
import functools
import math

import jax
import jax.numpy as jnp
from jax import lax
from jax.experimental import pallas as pl
from jax.experimental.pallas import tpu as pltpu
from jax.experimental.pallas import tpu_sc as plsc

_PCT = 0.1
_NC, _NS, _L = 2, 16, 16
_NW = _NC * _NS
_INT_MIN = -(2 ** 31)
_NSEG = 8


def _gat(v, idx):
    return jnp.take_along_axis(v, idx, axis=0)


def _sc_body(n_feat, k, xbits, thr_out, rowbuf, bufa, hist, thrk):
    int_min = jnp.int32(_INT_MIN)
    lane = jnp.arange(_L, dtype=jnp.int32)
    ones = jnp.ones((_L,), jnp.int32)
    zeros16 = jnp.zeros((_L,), jnp.int32)
    last_idx = jnp.full((_L,), _L - 1, jnp.int32)
    nchunk = n_feat // _L
    rows_per_w = xbits.shape[0] // _NW
    seg_elems = n_feat // _NSEG

    cid = lax.axis_index("c")
    sid = lax.axis_index("s")
    wid = sid * _NC + cid

    def zero_hist():
        for j in range(16):
            hist[pl.ds(j * _L, _L)] = zeros16

    def scan(kprime_v):
        def it(c, carry):
            acc_v, found_v, brev_v, j0f_v, cgef_v, accf_v = carry
            t = hist[pl.ds(c * _L, _L)]
            cs = plsc.cumsum(t)
            cge = acc_v + cs
            m = cge >= kprime_v
            pc = plsc.all_reduce_population_count(m)
            j0 = 16 - pc
            fh = jnp.logical_and(found_v == 0, pc > 0)
            brev_v = jnp.where(fh, c * _L + j0, brev_v)
            j0f_v = jnp.where(fh, j0, j0f_v)
            cgef_v = jnp.where(fh, cge, cgef_v)
            accf_v = jnp.where(fh, acc_v, accf_v)
            found_v = jnp.where(pc > 0, jnp.int32(1), found_v)
            acc_v = acc_v + _gat(cs, last_idx)
            return acc_v, found_v, brev_v, j0f_v, cgef_v, accf_v
        init = (zeros16, zeros16, zeros16, zeros16, zeros16, zeros16)
        _, _, brev_v, j0f_v, cgef_v, accf_v = lax.fori_loop(0, 16, it, init)
        cnt_gt_v = jnp.where(j0f_v == 0, accf_v,
                             _gat(cgef_v, jnp.maximum(j0f_v - 1, 0)))
        t_b_v = _gat(cgef_v, j0f_v) - cnt_gt_v
        return brev_v, cnt_gt_v, t_b_v

    def row_fn(r, carry):
        row = wid * rows_per_w + r
        pltpu.sync_copy(xbits.at[row], rowbuf)
        kprime_v = jnp.full((_L,), k, jnp.int32)

        zero_hist()

        def sweep_a(i, c):
            for s in range(8):
                off = i * (8 * _L) + s * _L
                v = rowbuf[pl.ds(off, _L)]
                mag = v & jnp.int32(0x7FFFFFFF)
                u = jnp.where(mag == 0, int_min,
                              jnp.where(v < 0, ~v, v | int_min))
                rowbuf[pl.ds(off, _L)] = u
                rb = lax.shift_right_logical(~u, 24)
                plsc.addupdate_scatter(hist, [rb], ones)
            return c
        lax.fori_loop(0, nchunk // 8, sweep_a, 0)
        b0rev_v, cnt_gt_v, tb_v = scan(kprime_v)
        kprime_v = kprime_v - cnt_gt_v

        zero_hist()

        def sweep_b(i, offs):
            new_offs = []
            for s in range(_NSEG):
                off_s = offs[s]
                src = s * seg_elems + i * _L
                v = rowbuf[pl.ds(src, _L)]
                nv = ~v
                rb0 = lax.shift_right_logical(nv, 24)
                take = rb0 == b0rev_v
                ti = take.astype(jnp.int32)
                pos = (s * seg_elems + off_s) + plsc.cumsum(ti) - ti
                plsc.store_scatter(bufa, [pos], v, mask=take)
                rb1 = lax.shift_right_logical(nv, 16) & jnp.int32(0xFF)
                plsc.addupdate_scatter(hist, [rb1], ones, mask=take)
                new_offs.append(off_s + jnp.sum(ti))
            return tuple(new_offs)
        offs = lax.fori_loop(0, seg_elems // _L, sweep_b,
                             tuple(jnp.int32(0) for _ in range(_NSEG)))
        b1rev_v, cnt_gt_v, tb_v = scan(kprime_v)
        kprime_v = kprime_v - cnt_gt_v
        n2 = jnp.max(tb_v)

        zero_hist()

        def seg_sweep(s, off_c):
            n_s = offs[s]

            def it(i, off_c):
                v = bufa[pl.ds(s * seg_elems + i * _L, _L)]
                valid = (i * _L + lane) < n_s
                nv = ~v
                rb1 = lax.shift_right_logical(nv, 16) & jnp.int32(0xFF)
                take = jnp.logical_and(valid, rb1 == b1rev_v)
                ti = take.astype(jnp.int32)
                pos = off_c + plsc.cumsum(ti) - ti
                plsc.store_scatter(rowbuf, [pos], v, mask=take)
                rb2 = lax.shift_right_logical(nv, 8) & jnp.int32(0xFF)
                plsc.addupdate_scatter(hist, [rb2], ones, mask=take)
                return off_c + jnp.sum(ti)
            return lax.fori_loop(0, (n_s + _L - 1) // _L, it, off_c)

        off_c = jnp.int32(0)
        for s in range(_NSEG):
            off_c = seg_sweep(s, off_c)
        b2rev_v, cnt_gt_v, tb_v = scan(kprime_v)
        kprime_v = kprime_v - cnt_gt_v

        zero_hist()

        def sweep_d(i, c):
            v = rowbuf[pl.ds(i * _L, _L)]
            valid = (i * _L + lane) < n2
            nv = ~v
            rb2 = lax.shift_right_logical(nv, 8) & jnp.int32(0xFF)
            take = jnp.logical_and(valid, rb2 == b2rev_v)
            rb3 = nv & jnp.int32(0xFF)
            plsc.addupdate_scatter(hist, [rb3], ones, mask=take)
            return c
        lax.fori_loop(0, (n2 + _L - 1) // _L, sweep_d, 0)
        b3rev_v, _, _ = scan(kprime_v)

        rev = (lax.shift_left(b0rev_v, 24) | lax.shift_left(b1rev_v, 16) |
               lax.shift_left(b2rev_v, 8) | b3rev_v)
        prefix_v = ~rev
        key_v = prefix_v ^ int_min
        fbits_v = jnp.where(key_v < 0, ~prefix_v, key_v)
        thrk[...] = jnp.where(lane == r, fbits_v, thrk[...])
        return carry

    lax.fori_loop(0, rows_per_w, row_fn, 0)
    pltpu.sync_copy(thrk, thr_out.at[wid])


def _mask_body(x_ref, t_ref, o_ref):
    x = x_ref[...]
    o_ref[...] = jnp.where(x >= t_ref[...], x, 0.0)


def kernel(x):
    n_rows, n_feat = x.shape
    k = max(1, math.ceil(n_feat * _PCT))
    xbits = lax.bitcast_convert_type(x, jnp.int32)

    mesh = plsc.VectorSubcoreMesh(core_axis_name="c", subcore_axis_name="s",
                                  num_cores=_NC, num_subcores=_NS)
    body = functools.partial(_sc_body, n_feat, k)
    thr2d = pl.kernel(
        body,
        out_type=jax.ShapeDtypeStruct((_NW, _L), jnp.int32),
        mesh=mesh,
        scratch_types=[
            pltpu.VMEM((n_feat,), jnp.int32),
            pltpu.VMEM((n_feat,), jnp.int32),
            pltpu.VMEM((256,), jnp.int32),
            pltpu.VMEM((_L,), jnp.int32),
        ],
        compiler_params=pltpu.CompilerParams(needs_layout_passes=False),
    )(xbits)

    rows_per_w = n_rows // _NW
    thr = lax.bitcast_convert_type(
        thr2d[:, :rows_per_w].reshape(n_rows, 1), jnp.float32)

    rb = 8
    return pl.pallas_call(
        _mask_body,
        grid=(n_rows // rb,),
        in_specs=[pl.BlockSpec((rb, n_feat), lambda i: (i, 0)),
                  pl.BlockSpec((rb, 1), lambda i: (i, 0))],
        out_specs=pl.BlockSpec((rb, n_feat), lambda i: (i, 0)),
        out_shape=jax.ShapeDtypeStruct(x.shape, x.dtype),
    )(x, thr)


# speedup vs baseline: 2.3038x; 2.0961x over previous
import functools
import math

import jax
import jax.numpy as jnp
from jax import lax
from jax.experimental import pallas as pl
from jax.experimental.pallas import tpu as pltpu
from jax.experimental.pallas import tpu_sc as plsc

_PCT = 0.1
_NC, _NS, _L = 2, 16, 16
_NW = _NC * _NS
_INT_MIN = -(2 ** 31)
_NSEG = 8


def _gat(v, idx):
    return jnp.take_along_axis(v, idx, axis=0)


def _sc_body(n_feat, k, xbits, thr_out, rowbuf, bufa, hist, thrk):
    int_min = jnp.int32(_INT_MIN)
    lane = jnp.arange(_L, dtype=jnp.int32)
    ones = jnp.ones((_L,), jnp.int32)
    zeros16 = jnp.zeros((_L,), jnp.int32)
    last_idx = jnp.full((_L,), _L - 1, jnp.int32)
    nchunk = n_feat // _L
    rows_per_w = xbits.shape[0] // _NW
    seg_elems = n_feat // _NSEG

    cid = lax.axis_index("c")
    sid = lax.axis_index("s")
    wid = sid * _NC + cid

    def zero_hist():
        for j in range(16):
            hist[pl.ds(j * _L, _L)] = zeros16

    def scan(kprime_v):
        def it(c, carry):
            acc_v, found_v, brev_v, j0f_v, cgef_v, accf_v = carry
            t = hist[pl.ds(c * _L, _L)]
            cs = plsc.cumsum(t)
            cge = acc_v + cs
            m = cge >= kprime_v
            pc = plsc.all_reduce_population_count(m)
            j0 = 16 - pc
            fh = jnp.logical_and(found_v == 0, pc > 0)
            brev_v = jnp.where(fh, c * _L + j0, brev_v)
            j0f_v = jnp.where(fh, j0, j0f_v)
            cgef_v = jnp.where(fh, cge, cgef_v)
            accf_v = jnp.where(fh, acc_v, accf_v)
            found_v = jnp.where(pc > 0, jnp.int32(1), found_v)
            acc_v = acc_v + _gat(cs, last_idx)
            return acc_v, found_v, brev_v, j0f_v, cgef_v, accf_v
        init = (zeros16, zeros16, zeros16, zeros16, zeros16, zeros16)
        _, _, brev_v, j0f_v, cgef_v, accf_v = lax.fori_loop(0, 16, it, init)
        cnt_gt_v = jnp.where(j0f_v == 0, accf_v,
                             _gat(cgef_v, jnp.maximum(j0f_v - 1, 0)))
        t_b_v = _gat(cgef_v, j0f_v) - cnt_gt_v
        return brev_v, cnt_gt_v, t_b_v

    def row_fn(r, carry):
        row = wid * rows_per_w + r
        pltpu.sync_copy(xbits.at[row], rowbuf)
        kprime_v = jnp.full((_L,), k, jnp.int32)

        zero_hist()

        @plsc.parallel_loop(0, nchunk, unroll=8)
        def _sweep_a(i):
            off = i * _L
            v = rowbuf[pl.ds(off, _L)]
            mag = v & jnp.int32(0x7FFFFFFF)
            u = jnp.where(mag == 0, int_min,
                          jnp.where(v < 0, ~v, v | int_min))
            bufa[pl.ds(off, _L)] = u
            rb = lax.shift_right_logical(~u, 24)
            plsc.addupdate_scatter(hist, [rb], ones)

        b0rev_v, cnt_gt_v, tb_v = scan(kprime_v)
        kprime_v = kprime_v - cnt_gt_v

        zero_hist()

        def _sweep_b(i, offs):
            new_offs = []
            for s in range(_NSEG):
                u = bufa[pl.ds(s * seg_elems + i * _L, _L)]
                rb0 = lax.shift_right_logical(~u, 24)
                take = rb0 == b0rev_v
                ti = take.astype(jnp.int32)
                cs = plsc.cumsum(ti)
                pos = (s * seg_elems) + offs[s] + cs - ti
                plsc.store_scatter(rowbuf, [pos], u, mask=take)
                rb1 = lax.shift_right_logical(~u, 16) & jnp.int32(0xFF)
                plsc.addupdate_scatter(hist, [rb1], ones, mask=take)
                new_offs.append(offs[s] + _gat(cs, last_idx))
            return tuple(new_offs)
        offs = plsc.parallel_loop(
            0, seg_elems // _L,
            carry=tuple(zeros16 for _ in range(_NSEG)))(_sweep_b)
        b1rev_v, cnt_gt_v, tb_v = scan(kprime_v)
        kprime_v = kprime_v - cnt_gt_v
        n2 = jnp.max(tb_v)

        zero_hist()
        off_c_v = zeros16
        for s in range(_NSEG):
            n_s_v = offs[s]
            n_s = jnp.max(n_s_v)

            def _seg_it(i, off_c_v, s=s, n_s_v=n_s_v):
                u = rowbuf[pl.ds(s * seg_elems + i * _L, _L)]
                valid = (i * _L + lane) < n_s_v
                rb1 = lax.shift_right_logical(~u, 16) & jnp.int32(0xFF)
                take = jnp.logical_and(valid, rb1 == b1rev_v)
                ti = take.astype(jnp.int32)
                cs = plsc.cumsum(ti)
                pos = off_c_v + cs - ti
                plsc.store_scatter(bufa, [pos], u, mask=take)
                rb2 = lax.shift_right_logical(~u, 8) & jnp.int32(0xFF)
                plsc.addupdate_scatter(hist, [rb2], ones, mask=take)
                return off_c_v + _gat(cs, last_idx)
            off_c_v = lax.fori_loop(0, (n_s + _L - 1) // _L, _seg_it,
                                    off_c_v)
        b2rev_v, cnt_gt_v, tb_v = scan(kprime_v)
        kprime_v = kprime_v - cnt_gt_v

        zero_hist()

        def _sweep_d(i, c):
            u = bufa[pl.ds(i * _L, _L)]
            valid = (i * _L + lane) < n2
            rb2 = lax.shift_right_logical(~u, 8) & jnp.int32(0xFF)
            take = jnp.logical_and(valid, rb2 == b2rev_v)
            rb3 = ~u & jnp.int32(0xFF)
            plsc.addupdate_scatter(hist, [rb3], ones, mask=take)
            return c
        lax.fori_loop(0, (n2 + _L - 1) // _L, _sweep_d, 0)
        b3rev_v, _, _ = scan(kprime_v)

        rev = (lax.shift_left(b0rev_v, 24) | lax.shift_left(b1rev_v, 16) |
               lax.shift_left(b2rev_v, 8) | b3rev_v)
        prefix_v = ~rev
        key_v = prefix_v ^ int_min
        fbits_v = jnp.where(key_v < 0, ~prefix_v, key_v)
        thrk[...] = jnp.where(lane == r, fbits_v, thrk[...])
        return carry

    lax.fori_loop(0, rows_per_w, row_fn, 0)
    pltpu.sync_copy(thrk, thr_out.at[wid])


def _mask_body(x_ref, t_ref, o_ref):
    x = x_ref[...]
    o_ref[...] = jnp.where(x >= t_ref[...], x, 0.0)


def kernel(x):
    n_rows, n_feat = x.shape
    k = max(1, math.ceil(n_feat * _PCT))
    xbits = lax.bitcast_convert_type(x, jnp.int32)

    mesh = plsc.VectorSubcoreMesh(core_axis_name="c", subcore_axis_name="s",
                                  num_cores=_NC, num_subcores=_NS)
    body = functools.partial(_sc_body, n_feat, k)
    thr2d = pl.kernel(
        body,
        out_type=jax.ShapeDtypeStruct((_NW, _L), jnp.int32),
        mesh=mesh,
        scratch_types=[
            pltpu.VMEM((n_feat,), jnp.int32),
            pltpu.VMEM((n_feat,), jnp.int32),
            pltpu.VMEM((256,), jnp.int32),
            pltpu.VMEM((_L,), jnp.int32),
        ],
        compiler_params=pltpu.CompilerParams(needs_layout_passes=False),
    )(xbits)

    rows_per_w = n_rows // _NW
    thr = lax.bitcast_convert_type(
        thr2d[:, :rows_per_w].reshape(n_rows, 1), jnp.float32)

    rb = 8
    return pl.pallas_call(
        _mask_body,
        grid=(n_rows // rb,),
        in_specs=[pl.BlockSpec((rb, n_feat), lambda i: (i, 0)),
                  pl.BlockSpec((rb, 1), lambda i: (i, 0))],
        out_specs=pl.BlockSpec((rb, n_feat), lambda i: (i, 0)),
        out_shape=jax.ShapeDtypeStruct(x.shape, x.dtype),
    )(x, thr)


# speedup vs baseline: 2.8070x; 1.2184x over previous
import functools
import math

import jax
import jax.numpy as jnp
from jax import lax
from jax.experimental import pallas as pl
from jax.experimental.pallas import tpu as pltpu
from jax.experimental.pallas import tpu_sc as plsc

_PCT = 0.1
_NC, _NS, _L = 2, 16, 16
_NW = _NC * _NS
_INT_MIN = -(2 ** 31)
_NSEG = 8


def _gat(v, idx):
    return jnp.take_along_axis(v, idx, axis=0)


def _sc_body(n_feat, k, xbits, thr_out, rowbuf, bufa, hist, thrk):
    int_min = jnp.int32(_INT_MIN)
    lane = jnp.arange(_L, dtype=jnp.int32)
    ones = jnp.ones((_L,), jnp.int32)
    zeros16 = jnp.zeros((_L,), jnp.int32)
    last_idx = jnp.full((_L,), _L - 1, jnp.int32)
    nchunk = n_feat // _L
    rows_per_w = xbits.shape[0] // _NW
    seg_elems = n_feat // _NSEG

    cid = lax.axis_index("c")
    sid = lax.axis_index("s")
    wid = sid * _NC + cid

    def zero_hist():
        for j in range(16):
            hist[pl.ds(j * _L, _L)] = zeros16

    def scan(kprime_v):
        def it(c, carry):
            acc_v, found_v, brev_v, j0f_v, cgef_v, accf_v = carry
            t = hist[pl.ds(c * _L, _L)]
            cs = plsc.cumsum(t)
            cge = acc_v + cs
            m = cge >= kprime_v
            pc = plsc.all_reduce_population_count(m)
            j0 = 16 - pc
            fh = jnp.logical_and(found_v == 0, pc > 0)
            brev_v = jnp.where(fh, c * _L + j0, brev_v)
            j0f_v = jnp.where(fh, j0, j0f_v)
            cgef_v = jnp.where(fh, cge, cgef_v)
            accf_v = jnp.where(fh, acc_v, accf_v)
            found_v = jnp.where(pc > 0, jnp.int32(1), found_v)
            acc_v = acc_v + _gat(cs, last_idx)
            return acc_v, found_v, brev_v, j0f_v, cgef_v, accf_v
        init = (zeros16, zeros16, zeros16, zeros16, zeros16, zeros16)
        _, _, brev_v, j0f_v, cgef_v, accf_v = lax.fori_loop(0, 16, it, init)
        cnt_gt_v = jnp.where(j0f_v == 0, accf_v,
                             _gat(cgef_v, jnp.maximum(j0f_v - 1, 0)))
        t_b_v = _gat(cgef_v, j0f_v) - cnt_gt_v
        return brev_v, cnt_gt_v, t_b_v

    def row_fn(r, carry):
        row = wid * rows_per_w + r
        pltpu.sync_copy(xbits.at[row], rowbuf)
        kprime_v = jnp.full((_L,), k, jnp.int32)

        zero_hist()

        @plsc.parallel_loop(0, nchunk, unroll=8)
        def _sweep_a(i):
            off = i * _L
            v = rowbuf[pl.ds(off, _L)]
            mag = v & jnp.int32(0x7FFFFFFF)
            u = jnp.where(mag == 0, int_min,
                          jnp.where(v < 0, ~v, v | int_min))
            bufa[pl.ds(off, _L)] = u
            rb = lax.shift_right_logical(~u, 24)
            plsc.addupdate_scatter(hist, [rb], ones)

        b0rev_v, cnt_gt_v, tb_v = scan(kprime_v)
        kprime_v = kprime_v - cnt_gt_v
        prefix_rev_v = b0rev_v

        for shift in (16, 8, 0):
            zero_hist()

            def _sweep(i, shift=shift, pfx=prefix_rev_v):
                u = bufa[pl.ds(i * _L, _L)]
                nv = ~u
                take = lax.shift_right_logical(nv, shift + 8) == pfx
                rb = lax.shift_right_logical(nv, shift) & jnp.int32(0xFF)
                plsc.addupdate_scatter(hist, [rb], ones, mask=take)
            plsc.parallel_loop(0, nchunk, unroll=8)(_sweep)
            brev_v, cnt_gt_v, tb_v = scan(kprime_v)
            kprime_v = kprime_v - cnt_gt_v
            prefix_rev_v = lax.shift_left(prefix_rev_v, 8) | brev_v

        prefix_v = ~prefix_rev_v
        key_v = prefix_v ^ int_min
        fbits_v = jnp.where(key_v < 0, ~prefix_v, key_v)
        thrk[...] = jnp.where(lane == r, fbits_v, thrk[...])
        return carry

    lax.fori_loop(0, rows_per_w, row_fn, 0)
    pltpu.sync_copy(thrk, thr_out.at[wid])


def _mask_body(x_ref, t_ref, o_ref):
    x = x_ref[...]
    o_ref[...] = jnp.where(x >= t_ref[...], x, 0.0)


def kernel(x):
    n_rows, n_feat = x.shape
    k = max(1, math.ceil(n_feat * _PCT))
    xbits = lax.bitcast_convert_type(x, jnp.int32)

    mesh = plsc.VectorSubcoreMesh(core_axis_name="c", subcore_axis_name="s",
                                  num_cores=_NC, num_subcores=_NS)
    body = functools.partial(_sc_body, n_feat, k)
    thr2d = pl.kernel(
        body,
        out_type=jax.ShapeDtypeStruct((_NW, _L), jnp.int32),
        mesh=mesh,
        scratch_types=[
            pltpu.VMEM((n_feat,), jnp.int32),
            pltpu.VMEM((n_feat,), jnp.int32),
            pltpu.VMEM((256,), jnp.int32),
            pltpu.VMEM((_L,), jnp.int32),
        ],
        compiler_params=pltpu.CompilerParams(needs_layout_passes=False),
    )(xbits)

    rows_per_w = n_rows // _NW
    thr = lax.bitcast_convert_type(
        thr2d[:, :rows_per_w].reshape(n_rows, 1), jnp.float32)

    rb = 8
    return pl.pallas_call(
        _mask_body,
        grid=(n_rows // rb,),
        in_specs=[pl.BlockSpec((rb, n_feat), lambda i: (i, 0)),
                  pl.BlockSpec((rb, 1), lambda i: (i, 0))],
        out_specs=pl.BlockSpec((rb, n_feat), lambda i: (i, 0)),
        out_shape=jax.ShapeDtypeStruct(x.shape, x.dtype),
    )(x, thr)


# speedup vs baseline: 2.8701x; 1.0225x over previous
import functools
import math

import jax
import jax.numpy as jnp
from jax import lax
from jax.experimental import pallas as pl
from jax.experimental.pallas import tpu as pltpu
from jax.experimental.pallas import tpu_sc as plsc

_PCT = 0.1
_NC, _NS, _L = 2, 16, 16
_NW = _NC * _NS
_INT_MIN = -(2 ** 31)


def _gat(v, idx):
    return jnp.take_along_axis(v, idx, axis=0)


def _sc_body(n_feat, k, xbits, out, rowbuf, bufa, bufc, hist,
             in_sem, out_sem):
    int_min = jnp.int32(_INT_MIN)
    lane = jnp.arange(_L, dtype=jnp.int32)
    ones = jnp.ones((_L,), jnp.int32)
    zeros16 = jnp.zeros((_L,), jnp.int32)
    last_idx = jnp.full((_L,), _L - 1, jnp.int32)
    nchunk = n_feat // _L
    rows_per_w = xbits.shape[0] // _NW

    cid = lax.axis_index("c")
    sid = lax.axis_index("s")
    wid = sid * _NC + cid
    row0 = wid * rows_per_w

    def zero_hist():
        for j in range(16):
            hist[pl.ds(j * _L, _L)] = zeros16

    def scan(kprime_v):
        def it(c, carry):
            acc_v, found_v, brev_v, j0f_v, cgef_v, accf_v = carry
            t = hist[pl.ds(c * _L, _L)]
            cs = plsc.cumsum(t)
            cge = acc_v + cs
            m = cge >= kprime_v
            pc = plsc.all_reduce_population_count(m)
            j0 = 16 - pc
            fh = jnp.logical_and(found_v == 0, pc > 0)
            brev_v = jnp.where(fh, c * _L + j0, brev_v)
            j0f_v = jnp.where(fh, j0, j0f_v)
            cgef_v = jnp.where(fh, cge, cgef_v)
            accf_v = jnp.where(fh, acc_v, accf_v)
            found_v = jnp.where(pc > 0, jnp.int32(1), found_v)
            acc_v = acc_v + _gat(cs, last_idx)
            return acc_v, found_v, brev_v, j0f_v, cgef_v, accf_v
        init = (zeros16, zeros16, zeros16, zeros16, zeros16, zeros16)
        _, _, brev_v, j0f_v, cgef_v, accf_v = lax.fori_loop(0, 16, it, init)
        cnt_gt_v = jnp.where(j0f_v == 0, accf_v,
                             _gat(cgef_v, jnp.maximum(j0f_v - 1, 0)))
        return brev_v, cnt_gt_v

    pltpu.sync_copy(xbits.at[row0], rowbuf)

    def row_fn(r, carry):
        row = row0 + r

        zero_hist()

        @plsc.parallel_loop(0, nchunk, unroll=8)
        def _sweep_a(i):
            off = i * _L
            v = rowbuf[pl.ds(off, _L)]
            mag = v & jnp.int32(0x7FFFFFFF)
            u = jnp.where(mag == 0, int_min,
                          jnp.where(v < 0, ~v, v | int_min))
            bufa[pl.ds(off, _L)] = u
            rb = lax.shift_right_logical(~u, 24)
            plsc.addupdate_scatter(hist, [rb], ones)

        @pl.when(r < rows_per_w - 1)
        def _():
            pltpu.make_async_copy(xbits.at[row + 1], rowbuf, in_sem).start()

        kprime_v = jnp.full((_L,), k, jnp.int32)
        brev_v, cnt_gt_v = scan(kprime_v)
        kprime_v = kprime_v - cnt_gt_v
        prefix_rev_v = brev_v

        for shift in (16, 8, 0):
            zero_hist()

            def _sweep(i, shift=shift, pfx=prefix_rev_v):
                u = bufa[pl.ds(i * _L, _L)]
                nv = ~u
                take = lax.shift_right_logical(nv, shift + 8) == pfx
                rb = lax.shift_right_logical(nv, shift) & jnp.int32(0xFF)
                plsc.addupdate_scatter(hist, [rb], ones, mask=take)
            plsc.parallel_loop(0, nchunk, unroll=8)(_sweep)
            brev_v, cnt_gt_v = scan(kprime_v)
            kprime_v = kprime_v - cnt_gt_v
            prefix_rev_v = lax.shift_left(prefix_rev_v, 8) | brev_v

        key_thr_v = ~prefix_rev_v ^ int_min

        @pl.when(r > 0)
        def _():
            pltpu.make_async_copy(bufc, out.at[row - 1], out_sem).wait()

        @plsc.parallel_loop(0, nchunk, unroll=8)
        def _sweep_m(i):
            off = i * _L
            u = bufa[pl.ds(off, _L)]
            key = u ^ int_min
            bits = jnp.where(key < 0, ~u, key)
            keep = key >= key_thr_v
            bufc[pl.ds(off, _L)] = jnp.where(keep, bits, jnp.int32(0))

        pltpu.make_async_copy(bufc, out.at[row], out_sem).start()

        @pl.when(r < rows_per_w - 1)
        def _():
            pltpu.make_async_copy(xbits.at[row + 1], rowbuf, in_sem).wait()
        return carry

    lax.fori_loop(0, rows_per_w, row_fn, 0)
    pltpu.make_async_copy(bufc, out.at[row0 + rows_per_w - 1],
                          out_sem).wait()


def kernel(x):
    n_rows, n_feat = x.shape
    k = max(1, math.ceil(n_feat * _PCT))
    xbits = lax.bitcast_convert_type(x, jnp.int32)

    mesh = plsc.VectorSubcoreMesh(core_axis_name="c", subcore_axis_name="s",
                                  num_cores=_NC, num_subcores=_NS)
    body = functools.partial(_sc_body, n_feat, k)
    out_i32 = pl.kernel(
        body,
        out_type=jax.ShapeDtypeStruct((n_rows, n_feat), jnp.int32),
        mesh=mesh,
        scratch_types=[
            pltpu.VMEM((n_feat,), jnp.int32),
            pltpu.VMEM((n_feat,), jnp.int32),
            pltpu.VMEM((n_feat,), jnp.int32),
            pltpu.VMEM((256,), jnp.int32),
            pltpu.SemaphoreType.DMA,
            pltpu.SemaphoreType.DMA,
        ],
        compiler_params=pltpu.CompilerParams(needs_layout_passes=False),
    )(xbits)
    return lax.bitcast_convert_type(out_i32, jnp.float32)
